# baseline (speedup 1.0000x reference)
import functools

import jax
import jax.numpy as jnp
from jax import lax
from jax.experimental import pallas as pl
from jax.experimental.pallas import tpu as pltpu
from jax.experimental.pallas import tpu_sc as plsc

N = 10000
R = 8
D = 128
E = 320000

NC = 2
NS = 16
NW = NC * NS
EW = E // NW
K = 128
NFULL = EW // K
KT = EW - NFULL * K
KA = 80
NCH = EW // KA
NTRI = 41
NCH3 = NTRI * 3
NR = N * R
NP = 10240
ROWS_T = NP // NS
DUMP = 16
DEG_T = NR // NS

_MESH = plsc.VectorSubcoreMesh(core_axis_name="c", subcore_axis_name="s")


def _wid(c, s):
    return s * NC + c


def _prep_body(src_hbm, dst_hbm, et_hbm,
               key_hbm, deg_hbm,
               sb, db, eb, kb, k2m, k2t, ones_b, zb, deg_sh, ssem):
    c = lax.axis_index("c")
    s = lax.axis_index("s")
    w = _wid(c, s)
    e0 = w * EW

    def zfill(i):
        zb[pl.ds(i * 16, 16)] = jnp.zeros((16,), jnp.float32)
    pl.loop(0, (DEG_T + 15) // 16)(zfill)
    pltpu.sync_copy(zb.at[pl.ds(0, DEG_T)],
                    deg_sh.at[pl.ds(s * DEG_T, DEG_T)])
    for g in range(K // 16):
        ones_b[pl.ds(g * 16, 16)] = jnp.ones((16,), jnp.float32)

    pltpu.sync_copy(src_hbm.at[pl.ds(e0, EW)], sb)
    pltpu.sync_copy(dst_hbm.at[pl.ds(e0, EW)], db)
    pltpu.sync_copy(et_hbm.at[pl.ds(e0, EW)], eb)
    plsc.subcore_barrier()

    def keys(g):
        sl = pl.ds(g * 16, 16)
        e = eb[sl]
        kb[sl] = e * N + sb[sl]
        row = g >> 3
        col = (g & 7) * 16
        k2m[row, pl.ds(col, 16)] = db[sl] * R + e
    pl.loop(0, NFULL * (K // 16))(keys)

    tl = pl.ds(NFULL * K, KT)
    kb[tl] = eb[tl] * N + sb[tl]
    k2t[...] = db[tl] * R + eb[tl]

    def fire(i):
        pltpu.async_copy(ones_b, deg_sh.at[k2m.at[i]], ssem, add=True)
    pl.loop(0, NFULL)(fire)
    pltpu.sync_copy(kb, key_hbm.at[pl.ds(e0, EW)])

    def drain(i):
        pltpu.make_async_copy(deg_hbm.at[pl.ds(0, K)], ones_b, ssem).wait()
    pl.loop(0, NFULL)(drain)
    pltpu.sync_copy(ones_b.at[pl.ds(0, KT)], deg_sh.at[k2t], add=True)

    plsc.subcore_barrier()
    pltpu.sync_copy(deg_sh.at[pl.ds(s * DEG_T, DEG_T)], zb.at[pl.ds(0, DEG_T)])
    pltpu.sync_copy(zb.at[pl.ds(0, DEG_T)],
                    deg_hbm.at[pl.ds(c * NR + s * DEG_T, DEG_T)])


_prep = pl.kernel(
    _prep_body,
    out_type=(
        jax.ShapeDtypeStruct((E,), jnp.int32),
        jax.ShapeDtypeStruct((NC * NR,), jnp.float32),
    ),
    mesh=_MESH,
    compiler_params=pltpu.CompilerParams(needs_layout_passes=False),
    scratch_types=(
        pltpu.VMEM((EW,), jnp.int32),
        pltpu.VMEM((EW,), jnp.int32),
        pltpu.VMEM((EW,), jnp.int32),
        pltpu.VMEM((EW,), jnp.int32),
        pltpu.VMEM((NFULL, K), jnp.int32),
        pltpu.VMEM((KT,), jnp.int32),
        pltpu.VMEM((K,), jnp.float32),
        pltpu.VMEM((((DEG_T + 15) // 16) * 16,), jnp.float32),
        pltpu.VMEM_SHARED((NR,), jnp.float32),
        pltpu.SemaphoreType.DMA,
    ),
)


def _norm_body(dst_hbm, et_hbm, recip_hbm, norm_hbm,
               db, eb, nb, recip_v):
    c = lax.axis_index("c")
    s = lax.axis_index("s")
    w = _wid(c, s)
    e0 = w * EW

    pltpu.sync_copy(recip_hbm, recip_v)
    pltpu.sync_copy(dst_hbm.at[pl.ds(e0, EW)], db)
    pltpu.sync_copy(et_hbm.at[pl.ds(e0, EW)], eb)

    def gath(g):
        sl = pl.ds(g * 16, 16)
        idx = db[sl] * R + eb[sl]
        nb[sl] = plsc.load_gather(recip_v, [idx])
    pl.loop(0, EW // 16)(gath)

    pltpu.sync_copy(nb, norm_hbm.at[pl.ds(e0, EW)])


_norm = pl.kernel(
    _norm_body,
    out_type=jax.ShapeDtypeStruct((E,), jnp.float32),
    mesh=_MESH,
    compiler_params=pltpu.CompilerParams(needs_layout_passes=False),
    scratch_types=(
        pltpu.VMEM((EW,), jnp.int32),
        pltpu.VMEM((EW,), jnp.int32),
        pltpu.VMEM((EW,), jnp.float32),
        pltpu.VMEM((NR,), jnp.float32),
    ),
)


def _agg_body(xw_hbm, key_hbm, dst_hbm, norm_hbm, acc_hbm,
              kc0, kc1, kc2, dc0, dc1, dc2, nc0, nc1, nc2,
              sx0, sx1, sx2, rows0, rows1, rows2, stg, acc_sh,
              gsem0, gsem1, gsem2, ssem0, ssem1, ssem2, isem):
    c = lax.axis_index("c")
    s = lax.axis_index("s")
    w = _wid(c, s)
    e0 = w * EW

    def zfill(i):
        for cc in range(D // 16):
            stg[i, pl.ds(cc * 16, 16)] = jnp.zeros((16,), jnp.float32)
    pl.loop(0, DUMP)(zfill)

    def zcopy(j):
        pltpu.sync_copy(stg, acc_sh.at[pl.ds(s * ROWS_T + j * DUMP, DUMP), :])
    pl.loop(0, ROWS_T // DUMP)(zcopy)
    plsc.subcore_barrier()

    kc = (kc0, kc1, kc2)
    dc = (dc0, dc1, dc2)
    nc = (nc0, nc1, nc2)
    sx = (sx0, sx1, sx2)
    rows = (rows0, rows1, rows2)
    gsem = (gsem0, gsem1, gsem2)
    ssem = (ssem0, ssem1, ssem2)

    def issue_idx(i, u):
        b = e0 + i * KA
        pltpu.async_copy(key_hbm.at[pl.ds(b, KA)], kc[u], isem)
        pltpu.async_copy(dst_hbm.at[pl.ds(b, KA)], dc[u], isem)
        pltpu.async_copy(norm_hbm.at[pl.ds(b, KA)], nc[u], isem)

    def wait_idx(u):
        pltpu.make_async_copy(key_hbm.at[pl.ds(0, KA)], kc[u], isem).wait()
        pltpu.make_async_copy(dst_hbm.at[pl.ds(0, KA)], dc[u], isem).wait()
        pltpu.make_async_copy(norm_hbm.at[pl.ds(0, KA)], nc[u], isem).wait()

    def issue_gather(u):
        pltpu.async_copy(xw_hbm.at[kc[u]], rows[u], gsem[u])

    def wait_gather(u):
        pltpu.make_async_copy(xw_hbm.at[pl.ds(0, KA), :], rows[u],
                              gsem[u]).wait()

    def issue_scatter(u):
        pltpu.async_copy(rows[u], acc_sh.at[sx[u]], ssem[u], add=True)

    def wait_scatter(u):
        pltpu.make_async_copy(rows[u], acc_sh.at[sx[u]], ssem[u]).wait()

    def scale(rows_ref, nrm_ref, g):
        nv = nrm_ref[pl.ds(g * 16, 16)]
        for l in range(16):
            j = g * 16 + l
            b = nv.at[jnp.full((16,), l, jnp.int32)].get(
                mode="promise_in_bounds")
            for cc in range(D // 16):
                sl = pl.ds(cc * 16, 16)
                rows_ref[j, sl] = rows_ref[j, sl] * b

    def body(i, u, pipelined):
        wait_gather(u)
        if pipelined:
            nxt = (u + 1) % 3

            @pl.when(i >= 2)
            def _():
                wait_scatter(nxt)

            @pl.when(i < NCH - 1)
            def _():
                wait_idx(nxt)
                issue_gather(nxt)
        pl.loop(0, KA // 16)(functools.partial(scale, rows[u], nc[u]))
        for g in range(KA // 16):
            sl = pl.ds(g * 16, 16)
            sx[u][sl] = dc[u][sl]
        if pipelined:
            issue_scatter(u)

            @pl.when(i < NCH - 3)
            def _():
                issue_idx(i + 3, u)
        else:
            pltpu.sync_copy(rows[u], acc_sh.at[sx[u]], add=True)

    issue_idx(0, 0)
    issue_idx(1, 1)
    issue_idx(2, 2)
    wait_idx(0)
    issue_gather(0)

    def triple(t):
        for u in range(3):
            body(t * 3 + u, u, True)

    pl.loop(0, NTRI)(triple)
    wait_scatter((NCH3 - 2) % 3)
    wait_scatter((NCH3 - 1) % 3)

    body(NCH3, NCH3 % 3, False)
    wait_idx((NCH3 + 1) % 3)
    issue_gather((NCH3 + 1) % 3)
    body(NCH3 + 1, (NCH3 + 1) % 3, False)

    plsc.subcore_barrier()

    def dump(j):
        r0 = s * ROWS_T + j * DUMP
        pltpu.sync_copy(acc_sh.at[pl.ds(r0, DUMP), :], stg)
        pltpu.sync_copy(stg, acc_hbm.at[c, pl.ds(r0, DUMP), :])
    pl.loop(0, ROWS_T // DUMP)(dump)


_agg = pl.kernel(
    _agg_body,
    out_type=jax.ShapeDtypeStruct((NC, NP, D), jnp.float32),
    mesh=_MESH,
    compiler_params=pltpu.CompilerParams(needs_layout_passes=False),
    scratch_types=(
        pltpu.VMEM((KA,), jnp.int32),
        pltpu.VMEM((KA,), jnp.int32),
        pltpu.VMEM((KA,), jnp.int32),
        pltpu.VMEM((KA,), jnp.int32),
        pltpu.VMEM((KA,), jnp.int32),
        pltpu.VMEM((KA,), jnp.int32),
        pltpu.VMEM((KA,), jnp.float32),
        pltpu.VMEM((KA,), jnp.float32),
        pltpu.VMEM((KA,), jnp.float32),
        pltpu.VMEM((KA,), jnp.int32),
        pltpu.VMEM((KA,), jnp.int32),
        pltpu.VMEM((KA,), jnp.int32),
        pltpu.VMEM((KA, D), jnp.float32),
        pltpu.VMEM((KA, D), jnp.float32),
        pltpu.VMEM((KA, D), jnp.float32),
        pltpu.VMEM((DUMP, D), jnp.float32),
        pltpu.VMEM_SHARED((NP, D), jnp.float32),
        pltpu.SemaphoreType.DMA,
        pltpu.SemaphoreType.DMA,
        pltpu.SemaphoreType.DMA,
        pltpu.SemaphoreType.DMA,
        pltpu.SemaphoreType.DMA,
        pltpu.SemaphoreType.DMA,
        pltpu.SemaphoreType.DMA,
    ),
)


BM = 1000
NBM = N // BM


def _einsum_body(x_ref, w_ref, o_ref):
    res = jnp.dot(x_ref[...], w_ref[...], preferred_element_type=jnp.float32)
    o_ref[...] = jnp.transpose(res.reshape(BM, R, D), (1, 0, 2))


def _einsum(x, wcat):
    return pl.pallas_call(
        _einsum_body,
        grid=(NBM,),
        in_specs=[
            pl.BlockSpec((BM, D), lambda j: (j, 0)),
            pl.BlockSpec((D, R * D), lambda j: (0, 0)),
        ],
        out_specs=pl.BlockSpec((R, BM, D), lambda j: (0, j, 0)),
        out_shape=jax.ShapeDtypeStruct((R, N, D), jnp.float32),
    )(x, wcat)


def _einsum_fused_body(p_ref, w_ref, o_ref):
    x = jnp.maximum(p_ref[0] + p_ref[1], 0.0)
    res = jnp.dot(x, w_ref[...], preferred_element_type=jnp.float32)
    o_ref[...] = jnp.transpose(res.reshape(BM, R, D), (1, 0, 2))


def _einsum_fused(pair, wcat):
    return pl.pallas_call(
        _einsum_fused_body,
        grid=(NBM,),
        in_specs=[
            pl.BlockSpec((2, BM, D), lambda j: (0, j, 0)),
            pl.BlockSpec((D, R * D), lambda j: (0, 0)),
        ],
        out_specs=pl.BlockSpec((R, BM, D), lambda j: (0, j, 0)),
        out_shape=jax.ShapeDtypeStruct((R, N, D), jnp.float32),
    )(pair, wcat)


def _combine(pair, op, bm, m=None):
    if m is None:
        m = pair.shape[1]

    def body(p_ref, o_ref):
        o_ref[...] = op(p_ref[0], p_ref[1])

    return pl.pallas_call(
        body,
        grid=(m // bm,),
        in_specs=[pl.BlockSpec((2, bm, D), lambda i: (0, i, 0))],
        out_specs=pl.BlockSpec((bm, D), lambda i: (i, 0)),
        out_shape=jax.ShapeDtypeStruct((m, D), jnp.float32),
    )(pair)


def _recip_op(a, b):
    return 1.0 / jnp.maximum(a + b, 1.0)


def _add_op(a, b):
    return a + b


@jax.jit
def kernel(X, edge_index, edge_type, W1, W2):
    src = edge_index[0]
    dst = edge_index[1]
    et = edge_type
    w1c = W1.transpose(1, 0, 2).reshape(D, R * D)
    w2c = W2.transpose(1, 0, 2).reshape(D, R * D)
    key, deg = _prep(src, dst, et)
    recip = _combine(deg.reshape(NC, NR // D, D), _recip_op, NR // D)
    norm = _norm(dst, et, recip.reshape(NR))

    xw1 = _einsum(X, w1c).reshape(R * N, D)
    acc1 = _agg(xw1, key, dst, norm)
    xw2 = _einsum_fused(acc1, w2c).reshape(R * N, D)
    acc2 = _agg(xw2, key, dst, norm)
    return _combine(acc2, _add_op, 1000, N)


# speedup vs baseline: 1.0014x; 1.0014x over previous
import functools

import jax
import jax.numpy as jnp
from jax import lax
from jax.experimental import pallas as pl
from jax.experimental.pallas import tpu as pltpu
from jax.experimental.pallas import tpu_sc as plsc

N = 10000
R = 8
D = 128
E = 320000

NC = 2
NS = 16
NW = NC * NS
EW = E // NW
K = 128
NFULL = EW // K
KT = EW - NFULL * K
KA = 80
NCH = EW // KA
NTRI = 41
NCH3 = NTRI * 3
NR = N * R
NP = 10240
ROWS_T = NP // NS
DUMP = 16
DEG_T = NR // NS

_MESH = plsc.VectorSubcoreMesh(core_axis_name="c", subcore_axis_name="s")


def _wid(c, s):
    return s * NC + c


def _prep_body(src_hbm, dst_hbm, et_hbm,
               key_hbm, deg_hbm,
               sb, db, eb, kb, k2m, k2t, ones_b, zb, deg_sh, ssem):
    c = lax.axis_index("c")
    s = lax.axis_index("s")
    w = _wid(c, s)
    e0 = w * EW

    def zfill(i):
        zb[pl.ds(i * 16, 16)] = jnp.zeros((16,), jnp.float32)
    pl.loop(0, (DEG_T + 15) // 16)(zfill)
    pltpu.sync_copy(zb.at[pl.ds(0, DEG_T)],
                    deg_sh.at[pl.ds(s * DEG_T, DEG_T)])
    for g in range(K // 16):
        ones_b[pl.ds(g * 16, 16)] = jnp.ones((16,), jnp.float32)

    pltpu.sync_copy(src_hbm.at[pl.ds(e0, EW)], sb)
    pltpu.sync_copy(dst_hbm.at[pl.ds(e0, EW)], db)
    pltpu.sync_copy(et_hbm.at[pl.ds(e0, EW)], eb)
    plsc.subcore_barrier()

    def keys(g):
        sl = pl.ds(g * 16, 16)
        e = eb[sl]
        kb[sl] = e * N + sb[sl]
        row = g >> 3
        col = (g & 7) * 16
        k2m[row, pl.ds(col, 16)] = db[sl] * R + e
    pl.loop(0, NFULL * (K // 16))(keys)

    tl = pl.ds(NFULL * K, KT)
    kb[tl] = eb[tl] * N + sb[tl]
    k2t[...] = db[tl] * R + eb[tl]

    def fire(i):
        pltpu.async_copy(ones_b, deg_sh.at[k2m.at[i]], ssem, add=True)
    pl.loop(0, NFULL)(fire)
    pltpu.sync_copy(kb, key_hbm.at[pl.ds(e0, EW)])

    def drain(i):
        pltpu.make_async_copy(deg_hbm.at[pl.ds(0, K)], ones_b, ssem).wait()
    pl.loop(0, NFULL)(drain)
    pltpu.sync_copy(ones_b.at[pl.ds(0, KT)], deg_sh.at[k2t], add=True)

    plsc.subcore_barrier()
    pltpu.sync_copy(deg_sh.at[pl.ds(s * DEG_T, DEG_T)], zb.at[pl.ds(0, DEG_T)])
    pltpu.sync_copy(zb.at[pl.ds(0, DEG_T)],
                    deg_hbm.at[pl.ds(c * NR + s * DEG_T, DEG_T)])


_prep = pl.kernel(
    _prep_body,
    out_type=(
        jax.ShapeDtypeStruct((E,), jnp.int32),
        jax.ShapeDtypeStruct((NC * NR,), jnp.float32),
    ),
    mesh=_MESH,
    compiler_params=pltpu.CompilerParams(needs_layout_passes=False),
    scratch_types=(
        pltpu.VMEM((EW,), jnp.int32),
        pltpu.VMEM((EW,), jnp.int32),
        pltpu.VMEM((EW,), jnp.int32),
        pltpu.VMEM((EW,), jnp.int32),
        pltpu.VMEM((NFULL, K), jnp.int32),
        pltpu.VMEM((KT,), jnp.int32),
        pltpu.VMEM((K,), jnp.float32),
        pltpu.VMEM((((DEG_T + 15) // 16) * 16,), jnp.float32),
        pltpu.VMEM_SHARED((NR,), jnp.float32),
        pltpu.SemaphoreType.DMA,
    ),
)


def _norm_body(dst_hbm, et_hbm, recip_hbm, norm_hbm,
               db, eb, nb, recip_v):
    c = lax.axis_index("c")
    s = lax.axis_index("s")
    w = _wid(c, s)
    e0 = w * EW

    pltpu.sync_copy(recip_hbm, recip_v)
    pltpu.sync_copy(dst_hbm.at[pl.ds(e0, EW)], db)
    pltpu.sync_copy(et_hbm.at[pl.ds(e0, EW)], eb)

    def gath(g):
        sl = pl.ds(g * 16, 16)
        idx = db[sl] * R + eb[sl]
        nb[sl] = plsc.load_gather(recip_v, [idx])
    pl.loop(0, EW // 16)(gath)

    pltpu.sync_copy(nb, norm_hbm.at[pl.ds(e0, EW)])


_norm = pl.kernel(
    _norm_body,
    out_type=jax.ShapeDtypeStruct((E,), jnp.float32),
    mesh=_MESH,
    compiler_params=pltpu.CompilerParams(needs_layout_passes=False),
    scratch_types=(
        pltpu.VMEM((EW,), jnp.int32),
        pltpu.VMEM((EW,), jnp.int32),
        pltpu.VMEM((EW,), jnp.float32),
        pltpu.VMEM((NR,), jnp.float32),
    ),
)


def _agg_body(xw_hbm, key_hbm, dst_hbm, norm_hbm, acc_hbm,
              kc0, kc1, kc2, dc0, dc1, dc2, nc0, nc1, nc2,
              sx0, sx1, sx2, rows0, rows1, rows2, stg, acc_sh,
              gsem0, gsem1, gsem2, ssem0, ssem1, ssem2, isem):
    c = lax.axis_index("c")
    s = lax.axis_index("s")
    w = _wid(c, s)
    e0 = w * EW

    def zfill(i):
        for cc in range(D // 16):
            stg[i, pl.ds(cc * 16, 16)] = jnp.zeros((16,), jnp.float32)
    pl.loop(0, DUMP)(zfill)

    def zcopy(j):
        pltpu.sync_copy(stg, acc_sh.at[pl.ds(s * ROWS_T + j * DUMP, DUMP), :])
    pl.loop(0, ROWS_T // DUMP)(zcopy)
    plsc.subcore_barrier()

    kc = (kc0, kc1, kc2)
    dc = (dc0, dc1, dc2)
    nc = (nc0, nc1, nc2)
    sx = (sx0, sx1, sx2)
    rows = (rows0, rows1, rows2)
    gsem = (gsem0, gsem1, gsem2)
    ssem = (ssem0, ssem1, ssem2)

    def issue_idx(i, u):
        b = e0 + i * KA
        pltpu.async_copy(key_hbm.at[pl.ds(b, KA)], kc[u], isem)
        pltpu.async_copy(dst_hbm.at[pl.ds(b, KA)], dc[u], isem)
        pltpu.async_copy(norm_hbm.at[pl.ds(b, KA)], nc[u], isem)

    def wait_idx(u):
        pltpu.make_async_copy(key_hbm.at[pl.ds(0, KA)], kc[u], isem).wait()
        pltpu.make_async_copy(dst_hbm.at[pl.ds(0, KA)], dc[u], isem).wait()
        pltpu.make_async_copy(norm_hbm.at[pl.ds(0, KA)], nc[u], isem).wait()

    H = KA // 2

    def issue_gather(u):
        pltpu.async_copy(xw_hbm.at[kc[u].at[pl.ds(0, H)]],
                         rows[u].at[pl.ds(0, H), :], gsem[u])
        pltpu.async_copy(xw_hbm.at[kc[u].at[pl.ds(H, H)]],
                         rows[u].at[pl.ds(H, H), :], gsem[u])

    def wait_gather(u):
        pltpu.make_async_copy(xw_hbm.at[pl.ds(0, KA), :], rows[u],
                              gsem[u]).wait()

    def issue_scatter(u):
        pltpu.async_copy(rows[u], acc_sh.at[sx[u]], ssem[u], add=True)

    def wait_scatter(u):
        pltpu.make_async_copy(rows[u], acc_sh.at[sx[u]], ssem[u]).wait()

    def scale(rows_ref, nrm_ref, g):
        nv = nrm_ref[pl.ds(g * 16, 16)]
        for l in range(16):
            j = g * 16 + l
            b = nv.at[jnp.full((16,), l, jnp.int32)].get(
                mode="promise_in_bounds")
            for cc in range(D // 16):
                sl = pl.ds(cc * 16, 16)
                rows_ref[j, sl] = rows_ref[j, sl] * b

    def body(i, u, pipelined):
        wait_gather(u)
        if pipelined:
            nxt = (u + 1) % 3

            @pl.when(i >= 2)
            def _():
                wait_scatter(nxt)

            @pl.when(i < NCH - 1)
            def _():
                wait_idx(nxt)
                issue_gather(nxt)
        pl.loop(0, KA // 16)(functools.partial(scale, rows[u], nc[u]))
        for g in range(KA // 16):
            sl = pl.ds(g * 16, 16)
            sx[u][sl] = dc[u][sl]
        if pipelined:
            issue_scatter(u)

            @pl.when(i < NCH - 3)
            def _():
                issue_idx(i + 3, u)
        else:
            pltpu.sync_copy(rows[u], acc_sh.at[sx[u]], add=True)

    issue_idx(0, 0)
    issue_idx(1, 1)
    issue_idx(2, 2)
    wait_idx(0)
    issue_gather(0)

    def triple(t):
        for u in range(3):
            body(t * 3 + u, u, True)

    pl.loop(0, NTRI)(triple)
    wait_scatter((NCH3 - 2) % 3)
    wait_scatter((NCH3 - 1) % 3)

    body(NCH3, NCH3 % 3, False)
    wait_idx((NCH3 + 1) % 3)
    issue_gather((NCH3 + 1) % 3)
    body(NCH3 + 1, (NCH3 + 1) % 3, False)

    plsc.subcore_barrier()

    def dump(j):
        r0 = s * ROWS_T + j * DUMP
        pltpu.sync_copy(acc_sh.at[pl.ds(r0, DUMP), :], stg)
        pltpu.sync_copy(stg, acc_hbm.at[c, pl.ds(r0, DUMP), :])
    pl.loop(0, ROWS_T // DUMP)(dump)


_agg = pl.kernel(
    _agg_body,
    out_type=jax.ShapeDtypeStruct((NC, NP, D), jnp.float32),
    mesh=_MESH,
    compiler_params=pltpu.CompilerParams(needs_layout_passes=False),
    scratch_types=(
        pltpu.VMEM((KA,), jnp.int32),
        pltpu.VMEM((KA,), jnp.int32),
        pltpu.VMEM((KA,), jnp.int32),
        pltpu.VMEM((KA,), jnp.int32),
        pltpu.VMEM((KA,), jnp.int32),
        pltpu.VMEM((KA,), jnp.int32),
        pltpu.VMEM((KA,), jnp.float32),
        pltpu.VMEM((KA,), jnp.float32),
        pltpu.VMEM((KA,), jnp.float32),
        pltpu.VMEM((KA,), jnp.int32),
        pltpu.VMEM((KA,), jnp.int32),
        pltpu.VMEM((KA,), jnp.int32),
        pltpu.VMEM((KA, D), jnp.float32),
        pltpu.VMEM((KA, D), jnp.float32),
        pltpu.VMEM((KA, D), jnp.float32),
        pltpu.VMEM((DUMP, D), jnp.float32),
        pltpu.VMEM_SHARED((NP, D), jnp.float32),
        pltpu.SemaphoreType.DMA,
        pltpu.SemaphoreType.DMA,
        pltpu.SemaphoreType.DMA,
        pltpu.SemaphoreType.DMA,
        pltpu.SemaphoreType.DMA,
        pltpu.SemaphoreType.DMA,
        pltpu.SemaphoreType.DMA,
    ),
)


BM = 1000
NBM = N // BM


def _einsum_body(x_ref, w_ref, o_ref):
    res = jnp.dot(x_ref[...], w_ref[...], preferred_element_type=jnp.float32)
    o_ref[...] = jnp.transpose(res.reshape(BM, R, D), (1, 0, 2))


def _einsum(x, wcat):
    return pl.pallas_call(
        _einsum_body,
        grid=(NBM,),
        in_specs=[
            pl.BlockSpec((BM, D), lambda j: (j, 0)),
            pl.BlockSpec((D, R * D), lambda j: (0, 0)),
        ],
        out_specs=pl.BlockSpec((R, BM, D), lambda j: (0, j, 0)),
        out_shape=jax.ShapeDtypeStruct((R, N, D), jnp.float32),
    )(x, wcat)


def _einsum_fused_body(p_ref, w_ref, o_ref):
    x = jnp.maximum(p_ref[0] + p_ref[1], 0.0)
    res = jnp.dot(x, w_ref[...], preferred_element_type=jnp.float32)
    o_ref[...] = jnp.transpose(res.reshape(BM, R, D), (1, 0, 2))


def _einsum_fused(pair, wcat):
    return pl.pallas_call(
        _einsum_fused_body,
        grid=(NBM,),
        in_specs=[
            pl.BlockSpec((2, BM, D), lambda j: (0, j, 0)),
            pl.BlockSpec((D, R * D), lambda j: (0, 0)),
        ],
        out_specs=pl.BlockSpec((R, BM, D), lambda j: (0, j, 0)),
        out_shape=jax.ShapeDtypeStruct((R, N, D), jnp.float32),
    )(pair, wcat)


def _combine(pair, op, bm, m=None):
    if m is None:
        m = pair.shape[1]

    def body(p_ref, o_ref):
        o_ref[...] = op(p_ref[0], p_ref[1])

    return pl.pallas_call(
        body,
        grid=(m // bm,),
        in_specs=[pl.BlockSpec((2, bm, D), lambda i: (0, i, 0))],
        out_specs=pl.BlockSpec((bm, D), lambda i: (i, 0)),
        out_shape=jax.ShapeDtypeStruct((m, D), jnp.float32),
    )(pair)


def _recip_op(a, b):
    return 1.0 / jnp.maximum(a + b, 1.0)


def _add_op(a, b):
    return a + b


@jax.jit
def kernel(X, edge_index, edge_type, W1, W2):
    src = edge_index[0]
    dst = edge_index[1]
    et = edge_type
    w1c = W1.transpose(1, 0, 2).reshape(D, R * D)
    w2c = W2.transpose(1, 0, 2).reshape(D, R * D)
    key, deg = _prep(src, dst, et)
    recip = _combine(deg.reshape(NC, NR // D, D), _recip_op, NR // D)
    norm = _norm(dst, et, recip.reshape(NR))

    xw1 = _einsum(X, w1c).reshape(R * N, D)
    acc1 = _agg(xw1, key, dst, norm)
    xw2 = _einsum_fused(acc1, w2c).reshape(R * N, D)
    acc2 = _agg(xw2, key, dst, norm)
    return _combine(acc2, _add_op, 1000, N)


# speedup vs baseline: 1.0023x; 1.0009x over previous
import functools

import jax
import jax.numpy as jnp
from jax import lax
from jax.experimental import pallas as pl
from jax.experimental.pallas import tpu as pltpu
from jax.experimental.pallas import tpu_sc as plsc

N = 10000
R = 8
D = 128
E = 320000

NC = 2
NS = 16
NW = NC * NS
EW = E // NW
K = 128
NFULL = EW // K
KT = EW - NFULL * K
NR = N * R
NP = 10240
ROWS_T = NP // NS
DUMP = 32
DEG_T = NR // NS

_MESH = plsc.VectorSubcoreMesh(core_axis_name="c", subcore_axis_name="s")


def _wid(c, s):
    return s * NC + c


def _prep_body(src_hbm, dst_hbm, et_hbm,
               key_hbm, deg_hbm,
               sb, db, eb, kb, k2m, k2t, ones_b, zb, deg_sh, ssem):
    c = lax.axis_index("c")
    s = lax.axis_index("s")
    w = _wid(c, s)
    e0 = w * EW

    def zfill(i):
        zb[pl.ds(i * 16, 16)] = jnp.zeros((16,), jnp.float32)
    pl.loop(0, (DEG_T + 15) // 16)(zfill)
    pltpu.sync_copy(zb.at[pl.ds(0, DEG_T)],
                    deg_sh.at[pl.ds(s * DEG_T, DEG_T)])
    for g in range(K // 16):
        ones_b[pl.ds(g * 16, 16)] = jnp.ones((16,), jnp.float32)

    pltpu.sync_copy(src_hbm.at[pl.ds(e0, EW)], sb)
    pltpu.sync_copy(dst_hbm.at[pl.ds(e0, EW)], db)
    pltpu.sync_copy(et_hbm.at[pl.ds(e0, EW)], eb)
    plsc.subcore_barrier()

    def keys(g):
        sl = pl.ds(g * 16, 16)
        e = eb[sl]
        kb[sl] = e * N + sb[sl]
        row = g >> 3
        col = (g & 7) * 16
        k2m[row, pl.ds(col, 16)] = db[sl] * R + e
    pl.loop(0, NFULL * (K // 16))(keys)

    tl = pl.ds(NFULL * K, KT)
    kb[tl] = eb[tl] * N + sb[tl]
    k2t[...] = db[tl] * R + eb[tl]

    def fire(i):
        pltpu.async_copy(ones_b, deg_sh.at[k2m.at[i]], ssem, add=True)
    pl.loop(0, NFULL)(fire)
    pltpu.sync_copy(kb, key_hbm.at[pl.ds(e0, EW)])

    def drain(i):
        pltpu.make_async_copy(deg_hbm.at[pl.ds(0, K)], ones_b, ssem).wait()
    pl.loop(0, NFULL)(drain)
    pltpu.sync_copy(ones_b.at[pl.ds(0, KT)], deg_sh.at[k2t], add=True)

    plsc.subcore_barrier()
    pltpu.sync_copy(deg_sh.at[pl.ds(s * DEG_T, DEG_T)], zb.at[pl.ds(0, DEG_T)])
    pltpu.sync_copy(zb.at[pl.ds(0, DEG_T)],
                    deg_hbm.at[pl.ds(c * NR + s * DEG_T, DEG_T)])


_prep = pl.kernel(
    _prep_body,
    out_type=(
        jax.ShapeDtypeStruct((E,), jnp.int32),
        jax.ShapeDtypeStruct((NC * NR,), jnp.float32),
    ),
    mesh=_MESH,
    compiler_params=pltpu.CompilerParams(needs_layout_passes=False),
    scratch_types=(
        pltpu.VMEM((EW,), jnp.int32),
        pltpu.VMEM((EW,), jnp.int32),
        pltpu.VMEM((EW,), jnp.int32),
        pltpu.VMEM((EW,), jnp.int32),
        pltpu.VMEM((NFULL, K), jnp.int32),
        pltpu.VMEM((KT,), jnp.int32),
        pltpu.VMEM((K,), jnp.float32),
        pltpu.VMEM((((DEG_T + 15) // 16) * 16,), jnp.float32),
        pltpu.VMEM_SHARED((NR,), jnp.float32),
        pltpu.SemaphoreType.DMA,
    ),
)


def _norm_body(dst_hbm, et_hbm, recip_hbm, norm_hbm,
               db, eb, nb, recip_v):
    c = lax.axis_index("c")
    s = lax.axis_index("s")
    w = _wid(c, s)
    e0 = w * EW

    pltpu.sync_copy(recip_hbm, recip_v)
    pltpu.sync_copy(dst_hbm.at[pl.ds(e0, EW)], db)
    pltpu.sync_copy(et_hbm.at[pl.ds(e0, EW)], eb)

    def gath(g):
        sl = pl.ds(g * 16, 16)
        idx = db[sl] * R + eb[sl]
        nb[sl] = plsc.load_gather(recip_v, [idx])
    pl.loop(0, EW // 16)(gath)

    pltpu.sync_copy(nb, norm_hbm.at[pl.ds(e0, EW)])


_norm = pl.kernel(
    _norm_body,
    out_type=jax.ShapeDtypeStruct((E,), jnp.float32),
    mesh=_MESH,
    compiler_params=pltpu.CompilerParams(needs_layout_passes=False),
    scratch_types=(
        pltpu.VMEM((EW,), jnp.int32),
        pltpu.VMEM((EW,), jnp.int32),
        pltpu.VMEM((EW,), jnp.float32),
        pltpu.VMEM((NR,), jnp.float32),
    ),
)


def _agg_body(xw_hbm, key_hbm, dst_hbm, norm_hbm, acc_hbm,
              kc0, kc1, dc0, dc1, nc0, nc1, rows0, rows1,
              kbt, dbt, nbt, rowst, stg, acc_sh,
              gsem0, gsem1, isem):
    c = lax.axis_index("c")
    s = lax.axis_index("s")
    w = _wid(c, s)
    e0 = w * EW

    def zfill(i):
        for cc in range(D // 16):
            stg[i, pl.ds(cc * 16, 16)] = jnp.zeros((16,), jnp.float32)
    pl.loop(0, DUMP)(zfill)

    def zcopy(j):
        pltpu.sync_copy(stg, acc_sh.at[pl.ds(s * ROWS_T + j * DUMP, DUMP), :])
    pl.loop(0, ROWS_T // DUMP)(zcopy)
    plsc.subcore_barrier()

    kc = (kc0, kc1)
    dc = (dc0, dc1)
    nc = (nc0, nc1)
    rows = (rows0, rows1)
    gsem = (gsem0, gsem1)

    def issue_idx(i, u):
        b = e0 + i * K
        pltpu.async_copy(key_hbm.at[pl.ds(b, K)], kc[u], isem)
        pltpu.async_copy(dst_hbm.at[pl.ds(b, K)], dc[u], isem)
        pltpu.async_copy(norm_hbm.at[pl.ds(b, K)], nc[u], isem)

    def wait_idx(u):
        pltpu.make_async_copy(key_hbm.at[pl.ds(0, K)], kc[u], isem).wait()
        pltpu.make_async_copy(dst_hbm.at[pl.ds(0, K)], dc[u], isem).wait()
        pltpu.make_async_copy(norm_hbm.at[pl.ds(0, K)], nc[u], isem).wait()

    def issue_gather(u):
        pltpu.async_copy(xw_hbm.at[kc[u]], rows[u], gsem[u])

    def wait_gather(u):
        pltpu.make_async_copy(xw_hbm.at[pl.ds(0, K), :], rows[u],
                              gsem[u]).wait()

    def scale(rows_ref, nrm_ref, g):
        nv = nrm_ref[pl.ds(g * 16, 16)]
        for l in range(16):
            j = g * 16 + l
            b = nv.at[jnp.full((16,), l, jnp.int32)].get(
                mode="promise_in_bounds")
            for cc in range(D // 16):
                sl = pl.ds(cc * 16, 16)
                rows_ref[j, sl] = rows_ref[j, sl] * b

    issue_idx(0, 0)
    wait_idx(0)
    issue_gather(0)
    issue_idx(1, 1)

    def pair(p):
        for u in range(2):
            i = p * 2 + u
            wait_gather(u)

            @pl.when(i < NFULL - 1)
            def _():
                wait_idx(1 - u)
                issue_gather(1 - u)

            pl.loop(0, K // 16)(functools.partial(scale, rows[u], nc[u]))
            pltpu.sync_copy(rows[u], acc_sh.at[dc[u]], add=True)

            @pl.when(i < NFULL - 2)
            def _():
                issue_idx(i + 2, u)

    pl.loop(0, NFULL // 2)(pair)

    b = e0 + NFULL * K
    pltpu.sync_copy(key_hbm.at[pl.ds(b, KT)], kbt)
    pltpu.sync_copy(dst_hbm.at[pl.ds(b, KT)], dbt)
    pltpu.sync_copy(norm_hbm.at[pl.ds(b, KT)], nbt)
    pltpu.async_copy(xw_hbm.at[kbt], rowst, gsem0)
    pltpu.make_async_copy(xw_hbm.at[pl.ds(0, KT), :], rowst, gsem0).wait()
    pl.loop(0, KT // 16)(functools.partial(scale, rowst, nbt))
    pltpu.sync_copy(rowst, acc_sh.at[dbt], add=True)

    plsc.subcore_barrier()

    def dump(j):
        r0 = s * ROWS_T + j * DUMP
        pltpu.sync_copy(acc_sh.at[pl.ds(r0, DUMP), :], stg)
        pltpu.sync_copy(stg, acc_hbm.at[c, pl.ds(r0, DUMP), :])
    pl.loop(0, ROWS_T // DUMP)(dump)


_agg = pl.kernel(
    _agg_body,
    out_type=jax.ShapeDtypeStruct((NC, NP, D), jnp.float32),
    mesh=_MESH,
    compiler_params=pltpu.CompilerParams(needs_layout_passes=False),
    scratch_types=(
        pltpu.VMEM((K,), jnp.int32),
        pltpu.VMEM((K,), jnp.int32),
        pltpu.VMEM((K,), jnp.int32),
        pltpu.VMEM((K,), jnp.int32),
        pltpu.VMEM((K,), jnp.float32),
        pltpu.VMEM((K,), jnp.float32),
        pltpu.VMEM((K, D), jnp.float32),
        pltpu.VMEM((K, D), jnp.float32),
        pltpu.VMEM((KT,), jnp.int32),
        pltpu.VMEM((KT,), jnp.int32),
        pltpu.VMEM((KT,), jnp.float32),
        pltpu.VMEM((KT, D), jnp.float32),
        pltpu.VMEM((DUMP, D), jnp.float32),
        pltpu.VMEM_SHARED((NP, D), jnp.float32),
        pltpu.SemaphoreType.DMA,
        pltpu.SemaphoreType.DMA,
        pltpu.SemaphoreType.DMA,
    ),
)


BM = 1000
NBM = N // BM


def _einsum_body(x_ref, w_ref, o_ref):
    res = jnp.dot(x_ref[...], w_ref[...], preferred_element_type=jnp.float32)
    o_ref[...] = jnp.transpose(res.reshape(BM, R, D), (1, 0, 2))


def _einsum(x, wcat):
    return pl.pallas_call(
        _einsum_body,
        grid=(NBM,),
        in_specs=[
            pl.BlockSpec((BM, D), lambda j: (j, 0)),
            pl.BlockSpec((D, R * D), lambda j: (0, 0)),
        ],
        out_specs=pl.BlockSpec((R, BM, D), lambda j: (0, j, 0)),
        out_shape=jax.ShapeDtypeStruct((R, N, D), jnp.float32),
    )(x, wcat)


def _einsum_fused_body(p_ref, w_ref, o_ref):
    x = jnp.maximum(p_ref[0] + p_ref[1], 0.0)
    res = jnp.dot(x, w_ref[...], preferred_element_type=jnp.float32)
    o_ref[...] = jnp.transpose(res.reshape(BM, R, D), (1, 0, 2))


def _einsum_fused(pair, wcat):
    return pl.pallas_call(
        _einsum_fused_body,
        grid=(NBM,),
        in_specs=[
            pl.BlockSpec((2, BM, D), lambda j: (0, j, 0)),
            pl.BlockSpec((D, R * D), lambda j: (0, 0)),
        ],
        out_specs=pl.BlockSpec((R, BM, D), lambda j: (0, j, 0)),
        out_shape=jax.ShapeDtypeStruct((R, N, D), jnp.float32),
    )(pair, wcat)


def _combine(pair, op, bm, m=None):
    if m is None:
        m = pair.shape[1]

    def body(p_ref, o_ref):
        o_ref[...] = op(p_ref[0], p_ref[1])

    return pl.pallas_call(
        body,
        grid=(m // bm,),
        in_specs=[pl.BlockSpec((2, bm, D), lambda i: (0, i, 0))],
        out_specs=pl.BlockSpec((bm, D), lambda i: (i, 0)),
        out_shape=jax.ShapeDtypeStruct((m, D), jnp.float32),
    )(pair)


def _recip_op(a, b):
    return 1.0 / jnp.maximum(a + b, 1.0)


def _add_op(a, b):
    return a + b


@jax.jit
def kernel(X, edge_index, edge_type, W1, W2):
    src = edge_index[0]
    dst = edge_index[1]
    et = edge_type
    w1c = W1.transpose(1, 0, 2).reshape(D, R * D)
    w2c = W2.transpose(1, 0, 2).reshape(D, R * D)
    key, deg = _prep(src, dst, et)
    recip = _combine(deg.reshape(NC, NR // D, D), _recip_op, NR // D)
    norm = _norm(dst, et, recip.reshape(NR))

    xw1 = _einsum(X, w1c).reshape(R * N, D)
    acc1 = _agg(xw1, key, dst, norm)
    xw2 = _einsum_fused(acc1, w2c).reshape(R * N, D)
    acc2 = _agg(xw2, key, dst, norm)
    return _combine(acc2, _add_op, 1000, N)


# speedup vs baseline: 1.0034x; 1.0012x over previous
import functools

import jax
import jax.numpy as jnp
from jax import lax
from jax.experimental import pallas as pl
from jax.experimental.pallas import tpu as pltpu
from jax.experimental.pallas import tpu_sc as plsc

N = 10000
R = 8
D = 128
E = 320000

NC = 2
NS = 16
NW = NC * NS
EW = E // NW
K = 128
NFULL = EW // K
KT = EW - NFULL * K
NR = N * R
NP = 10240
ROWS_T = NP // NS
DUMP = 32
DEG_T = NR // NS

_MESH = plsc.VectorSubcoreMesh(core_axis_name="c", subcore_axis_name="s",
                               num_cores=NC, num_subcores=NS)


def _wid(c, s):
    return s * NC + c


def _prep_body(src_hbm, dst_hbm, et_hbm,
               key_hbm, deg_hbm,
               sb, db, eb, kb, k2m, k2t, ones_b, zb, deg_sh, ssem):
    c = lax.axis_index("c")
    s = lax.axis_index("s")
    w = _wid(c, s)
    e0 = w * EW

    def zfill(i):
        zb[pl.ds(i * 16, 16)] = jnp.zeros((16,), jnp.float32)
    pl.loop(0, (DEG_T + 15) // 16)(zfill)
    pltpu.sync_copy(zb.at[pl.ds(0, DEG_T)],
                    deg_sh.at[pl.ds(s * DEG_T, DEG_T)])
    for g in range(K // 16):
        ones_b[pl.ds(g * 16, 16)] = jnp.ones((16,), jnp.float32)

    pltpu.sync_copy(src_hbm.at[pl.ds(e0, EW)], sb)
    pltpu.sync_copy(dst_hbm.at[pl.ds(e0, EW)], db)
    pltpu.sync_copy(et_hbm.at[pl.ds(e0, EW)], eb)
    plsc.subcore_barrier()

    def keys(g):
        sl = pl.ds(g * 16, 16)
        e = eb[sl]
        kb[sl] = e * N + sb[sl]
        row = g >> 3
        col = (g & 7) * 16
        k2m[row, pl.ds(col, 16)] = db[sl] * R + e
    pl.loop(0, NFULL * (K // 16))(keys)

    tl = pl.ds(NFULL * K, KT)
    kb[tl] = eb[tl] * N + sb[tl]
    k2t[...] = db[tl] * R + eb[tl]

    def fire(i):
        pltpu.async_copy(ones_b, deg_sh.at[k2m.at[i]], ssem, add=True)
    pl.loop(0, NFULL)(fire)
    pltpu.sync_copy(kb, key_hbm.at[pl.ds(e0, EW)])

    def drain(i):
        pltpu.make_async_copy(deg_hbm.at[pl.ds(0, K)], ones_b, ssem).wait()
    pl.loop(0, NFULL)(drain)
    pltpu.sync_copy(ones_b.at[pl.ds(0, KT)], deg_sh.at[k2t], add=True)

    plsc.subcore_barrier()
    pltpu.sync_copy(deg_sh.at[pl.ds(s * DEG_T, DEG_T)], zb.at[pl.ds(0, DEG_T)])
    pltpu.sync_copy(zb.at[pl.ds(0, DEG_T)],
                    deg_hbm.at[pl.ds(c * NR + s * DEG_T, DEG_T)])


_prep = pl.kernel(
    _prep_body,
    out_type=(
        jax.ShapeDtypeStruct((E,), jnp.int32),
        jax.ShapeDtypeStruct((NC * NR,), jnp.float32),
    ),
    mesh=_MESH,
    compiler_params=pltpu.CompilerParams(needs_layout_passes=False),
    scratch_types=(
        pltpu.VMEM((EW,), jnp.int32),
        pltpu.VMEM((EW,), jnp.int32),
        pltpu.VMEM((EW,), jnp.int32),
        pltpu.VMEM((EW,), jnp.int32),
        pltpu.VMEM((NFULL, K), jnp.int32),
        pltpu.VMEM((KT,), jnp.int32),
        pltpu.VMEM((K,), jnp.float32),
        pltpu.VMEM((((DEG_T + 15) // 16) * 16,), jnp.float32),
        pltpu.VMEM_SHARED((NR,), jnp.float32),
        pltpu.SemaphoreType.DMA,
    ),
)


def _norm_body(dst_hbm, et_hbm, recip_hbm, norm_hbm,
               db, eb, nb, recip_v):
    c = lax.axis_index("c")
    s = lax.axis_index("s")
    w = _wid(c, s)
    e0 = w * EW

    pltpu.sync_copy(recip_hbm, recip_v)
    pltpu.sync_copy(dst_hbm.at[pl.ds(e0, EW)], db)
    pltpu.sync_copy(et_hbm.at[pl.ds(e0, EW)], eb)

    def gath(g):
        sl = pl.ds(g * 16, 16)
        idx = db[sl] * R + eb[sl]
        nb[sl] = plsc.load_gather(recip_v, [idx])
    pl.loop(0, EW // 16)(gath)

    pltpu.sync_copy(nb, norm_hbm.at[pl.ds(e0, EW)])


_norm = pl.kernel(
    _norm_body,
    out_type=jax.ShapeDtypeStruct((E,), jnp.float32),
    mesh=_MESH,
    compiler_params=pltpu.CompilerParams(needs_layout_passes=False),
    scratch_types=(
        pltpu.VMEM((EW,), jnp.int32),
        pltpu.VMEM((EW,), jnp.int32),
        pltpu.VMEM((EW,), jnp.float32),
        pltpu.VMEM((NR,), jnp.float32),
    ),
)


def _agg_body(xw_hbm, key_hbm, dst_hbm, norm_hbm, acc_hbm,
              kc0, kc1, dc0, dc1, nc0, nc1, rows0, rows1,
              kbt, dbt, nbt, rowst, stg, acc_sh,
              gsem0, gsem1, isem):
    c = lax.axis_index("c")
    s = lax.axis_index("s")
    w = _wid(c, s)
    e0 = w * EW

    def zfill(i):
        for cc in range(D // 16):
            stg[i, pl.ds(cc * 16, 16)] = jnp.zeros((16,), jnp.float32)
    pl.loop(0, DUMP)(zfill)

    def zcopy(j):
        pltpu.sync_copy(stg, acc_sh.at[pl.ds(s * ROWS_T + j * DUMP, DUMP), :])
    pl.loop(0, ROWS_T // DUMP)(zcopy)
    plsc.subcore_barrier()

    kc = (kc0, kc1)
    dc = (dc0, dc1)
    nc = (nc0, nc1)
    rows = (rows0, rows1)
    gsem = (gsem0, gsem1)

    def issue_idx(i, u):
        b = e0 + i * K
        pltpu.async_copy(key_hbm.at[pl.ds(b, K)], kc[u], isem)
        pltpu.async_copy(dst_hbm.at[pl.ds(b, K)], dc[u], isem)
        pltpu.async_copy(norm_hbm.at[pl.ds(b, K)], nc[u], isem)

    def wait_idx(u):
        pltpu.make_async_copy(key_hbm.at[pl.ds(0, K)], kc[u], isem).wait()
        pltpu.make_async_copy(dst_hbm.at[pl.ds(0, K)], dc[u], isem).wait()
        pltpu.make_async_copy(norm_hbm.at[pl.ds(0, K)], nc[u], isem).wait()

    def issue_gather(u):
        pltpu.async_copy(xw_hbm.at[kc[u]], rows[u], gsem[u])

    def wait_gather(u):
        pltpu.make_async_copy(xw_hbm.at[pl.ds(0, K), :], rows[u],
                              gsem[u]).wait()

    def scale(rows_ref, nrm_ref, g):
        nv = nrm_ref[pl.ds(g * 16, 16)]
        for l in range(16):
            j = g * 16 + l
            b = nv.at[jnp.full((16,), l, jnp.int32)].get(
                mode="promise_in_bounds")
            for cc in range(D // 16):
                sl = pl.ds(cc * 16, 16)
                rows_ref[j, sl] = rows_ref[j, sl] * b

    issue_idx(0, 0)
    wait_idx(0)
    issue_gather(0)
    issue_idx(1, 1)

    def pair(p):
        for u in range(2):
            i = p * 2 + u
            wait_gather(u)

            @pl.when(i < NFULL - 1)
            def _():
                wait_idx(1 - u)
                issue_gather(1 - u)

            pl.loop(0, K // 16)(functools.partial(scale, rows[u], nc[u]))
            pltpu.sync_copy(rows[u], acc_sh.at[dc[u]], add=True)

            @pl.when(i < NFULL - 2)
            def _():
                issue_idx(i + 2, u)

    pl.loop(0, NFULL // 2)(pair)

    b = e0 + NFULL * K
    pltpu.sync_copy(key_hbm.at[pl.ds(b, KT)], kbt)
    pltpu.sync_copy(dst_hbm.at[pl.ds(b, KT)], dbt)
    pltpu.sync_copy(norm_hbm.at[pl.ds(b, KT)], nbt)
    pltpu.async_copy(xw_hbm.at[kbt], rowst, gsem0)
    pltpu.make_async_copy(xw_hbm.at[pl.ds(0, KT), :], rowst, gsem0).wait()
    pl.loop(0, KT // 16)(functools.partial(scale, rowst, nbt))
    pltpu.sync_copy(rowst, acc_sh.at[dbt], add=True)

    plsc.subcore_barrier()

    def dump(j):
        r0 = s * ROWS_T + j * DUMP
        pltpu.sync_copy(acc_sh.at[pl.ds(r0, DUMP), :], stg)
        pltpu.sync_copy(stg, acc_hbm.at[c, pl.ds(r0, DUMP), :])
    pl.loop(0, ROWS_T // DUMP)(dump)


_agg = pl.kernel(
    _agg_body,
    out_type=jax.ShapeDtypeStruct((NC, NP, D), jnp.float32),
    mesh=_MESH,
    compiler_params=pltpu.CompilerParams(needs_layout_passes=False),
    scratch_types=(
        pltpu.VMEM((K,), jnp.int32),
        pltpu.VMEM((K,), jnp.int32),
        pltpu.VMEM((K,), jnp.int32),
        pltpu.VMEM((K,), jnp.int32),
        pltpu.VMEM((K,), jnp.float32),
        pltpu.VMEM((K,), jnp.float32),
        pltpu.VMEM((K, D), jnp.float32),
        pltpu.VMEM((K, D), jnp.float32),
        pltpu.VMEM((KT,), jnp.int32),
        pltpu.VMEM((KT,), jnp.int32),
        pltpu.VMEM((KT,), jnp.float32),
        pltpu.VMEM((KT, D), jnp.float32),
        pltpu.VMEM((DUMP, D), jnp.float32),
        pltpu.VMEM_SHARED((NP, D), jnp.float32),
        pltpu.SemaphoreType.DMA,
        pltpu.SemaphoreType.DMA,
        pltpu.SemaphoreType.DMA,
    ),
)


BM = 1000
NBM = N // BM


def _einsum_body(x_ref, w_ref, o_ref):
    res = jnp.dot(x_ref[...], w_ref[...], preferred_element_type=jnp.float32)
    o_ref[...] = jnp.transpose(res.reshape(BM, R, D), (1, 0, 2))


def _einsum(x, wcat):
    return pl.pallas_call(
        _einsum_body,
        grid=(NBM,),
        in_specs=[
            pl.BlockSpec((BM, D), lambda j: (j, 0)),
            pl.BlockSpec((D, R * D), lambda j: (0, 0)),
        ],
        out_specs=pl.BlockSpec((R, BM, D), lambda j: (0, j, 0)),
        out_shape=jax.ShapeDtypeStruct((R, N, D), jnp.float32),
    )(x, wcat)


def _einsum_fused_body(p_ref, w_ref, o_ref):
    x = jnp.maximum(p_ref[0] + p_ref[1], 0.0)
    res = jnp.dot(x, w_ref[...], preferred_element_type=jnp.float32)
    o_ref[...] = jnp.transpose(res.reshape(BM, R, D), (1, 0, 2))


def _einsum_fused(pair, wcat):
    return pl.pallas_call(
        _einsum_fused_body,
        grid=(NBM,),
        in_specs=[
            pl.BlockSpec((2, BM, D), lambda j: (0, j, 0)),
            pl.BlockSpec((D, R * D), lambda j: (0, 0)),
        ],
        out_specs=pl.BlockSpec((R, BM, D), lambda j: (0, j, 0)),
        out_shape=jax.ShapeDtypeStruct((R, N, D), jnp.float32),
    )(pair, wcat)


def _combine(pair, op, bm, m=None):
    if m is None:
        m = pair.shape[1]

    def body(p_ref, o_ref):
        o_ref[...] = op(p_ref[0], p_ref[1])

    return pl.pallas_call(
        body,
        grid=(m // bm,),
        in_specs=[pl.BlockSpec((2, bm, D), lambda i: (0, i, 0))],
        out_specs=pl.BlockSpec((bm, D), lambda i: (i, 0)),
        out_shape=jax.ShapeDtypeStruct((m, D), jnp.float32),
    )(pair)


def _recip_op(a, b):
    return 1.0 / jnp.maximum(a + b, 1.0)


def _add_op(a, b):
    return a + b


@jax.jit
def kernel(X, edge_index, edge_type, W1, W2):
    src = edge_index[0]
    dst = edge_index[1]
    et = edge_type
    w1c = W1.transpose(1, 0, 2).reshape(D, R * D)
    w2c = W2.transpose(1, 0, 2).reshape(D, R * D)
    key, deg = _prep(src, dst, et)
    recip = _combine(deg.reshape(NC, NR // D, D), _recip_op, NR // D)
    norm = _norm(dst, et, recip.reshape(NR))

    xw1 = _einsum(X, w1c).reshape(R * N, D)
    acc1 = _agg(xw1, key, dst, norm)
    xw2 = _einsum_fused(acc1, w2c).reshape(R * N, D)
    acc2 = _agg(xw2, key, dst, norm)
    return _combine(acc2, _add_op, 1000, N)
